# baseline (speedup 1.0000x reference)
import functools

import jax
import jax.numpy as jnp
from jax import lax
from jax.experimental import pallas as pl
from jax.experimental.pallas import tpu as pltpu
from jax.experimental.pallas import tpu_sc as plsc

N = 10000
D = 128
NC = 2
NS = 16
NW = NC * NS
CHUNK = 128
N_R = 10112
STRIPE = N_R // NS


def _sc_degree(dst3, z128, ones128, n_chunks):
    mesh = plsc.VectorSubcoreMesh(core_axis_name="c", subcore_axis_name="s")

    @functools.partial(
        pl.kernel,
        mesh=mesh,
        out_type=jax.ShapeDtypeStruct((NC, N_R, D), jnp.float32),
        scratch_types=[
            pltpu.VMEM((n_chunks, CHUNK), jnp.int32),
            pltpu.VMEM((CHUNK, D), jnp.float32),
            pltpu.VMEM_SHARED((N_R, D), jnp.float32),
        ],
    )
    def deg_k(dst_hbm, z128_hbm, ones_hbm, d_hbm, dst_v, ones_v, deg_s):
        c = lax.axis_index("c")
        s = lax.axis_index("s")
        wid = c * NS + s
        pltpu.sync_copy(dst_hbm.at[wid], dst_v)
        pltpu.sync_copy(ones_hbm, ones_v)
        row0 = s * STRIPE
        pltpu.sync_copy(z128_hbm, deg_s.at[pl.ds(row0, STRIPE)])
        plsc.subcore_barrier()

        def body(j, carry):
            pltpu.sync_copy(ones_v, deg_s.at[dst_v.at[j]], add=True)
            return carry

        lax.fori_loop(0, n_chunks, body, 0)
        plsc.subcore_barrier()
        pltpu.sync_copy(deg_s.at[pl.ds(row0, STRIPE)],
                        d_hbm.at[c].at[pl.ds(row0, STRIPE)])

    return deg_k(dst3, z128, ones128)


def _sc_aggregate(h, src3, dst3, z128, n_chunks):
    mesh = plsc.VectorSubcoreMesh(core_axis_name="c", subcore_axis_name="s")
    n_win = n_chunks // 8

    @functools.partial(
        pl.kernel,
        mesh=mesh,
        out_type=jax.ShapeDtypeStruct((NC, N_R, D), jnp.float32),
        scratch_types=[
            pltpu.VMEM((2, 8, CHUNK), jnp.int32),
            pltpu.VMEM((2, 8, CHUNK), jnp.int32),
            pltpu.VMEM((CHUNK, D), jnp.float32),
            pltpu.VMEM((CHUNK, D), jnp.float32),
            pltpu.VMEM_SHARED((N_R, D), jnp.float32),
            pltpu.SemaphoreType.DMA,
            pltpu.SemaphoreType.DMA,
            pltpu.SemaphoreType.DMA,
            pltpu.SemaphoreType.DMA,
            pltpu.SemaphoreType.DMA,
        ],
    )
    def agg(h_hbm, src_hbm, dst_hbm, z128_hbm, p_hbm,
            srcw, dstw, buf_a, buf_b, acc_s, sga, sgb, ssa, ssb, swin):
        c = lax.axis_index("c")
        s = lax.axis_index("s")
        wid = c * NS + s
        row0 = s * STRIPE
        pltpu.sync_copy(z128_hbm, acc_s.at[pl.ds(row0, STRIPE)])

        def fetch_win(par, w):
            pltpu.async_copy(src_hbm.at[wid].at[pl.ds(8 * w, 8)],
                             srcw.at[par], swin)
            pltpu.async_copy(dst_hbm.at[wid].at[pl.ds(8 * w, 8)],
                             dstw.at[par], swin)

        def fetch_wait(par, w):
            pltpu.make_async_copy(src_hbm.at[wid].at[pl.ds(8 * w, 8)],
                                  srcw.at[par], swin).wait()
            pltpu.make_async_copy(dst_hbm.at[wid].at[pl.ds(8 * w, 8)],
                                  dstw.at[par], swin).wait()

        def gath(buf, sem, par, k):
            pltpu.async_copy(h_hbm.at[srcw.at[par, k]], buf, sem)

        def gath_wait(buf, sem, par, k):
            pltpu.make_async_copy(h_hbm.at[srcw.at[par, k]], buf, sem).wait()

        def scat(buf, sem, par, k):
            pltpu.async_copy(buf, acc_s.at[dstw.at[par, k]], sem, add=True)

        def scat_wait(buf, sem, par, k):
            pltpu.make_async_copy(buf, acc_s.at[dstw.at[par, k]], sem).wait()

        fetch_win(0, 0)
        fetch_wait(0, 0)
        plsc.subcore_barrier()
        gath(buf_a, sga, 0, 0)

        def body(i, carry):
            fetch_win(1, 2 * i + 1)
            for m in range(8):
                par, ka = (0, 2 * m) if m < 4 else (1, 2 * m - 8)
                kb = ka + 1
                par2, k2 = (par, ka + 2) if ka < 6 else (1 - par, ka - 6)

                if m == 3:
                    fetch_wait(1, 2 * i + 1)

                if m == 0:
                    @pl.when(i > 0)
                    def _():
                        scat_wait(buf_b, ssb, 0, 7)
                else:
                    scat_wait(buf_b, ssb, par if kb - 2 >= 0 else 0,
                              kb - 2 if kb - 2 >= 0 else 7)

                gath(buf_b, sgb, par, kb)
                gath_wait(buf_a, sga, par, ka)
                scat(buf_a, ssa, par, ka)
                scat_wait(buf_a, ssa, par, ka)

                if m == 7:
                    @pl.when(2 * i + 2 < n_win)
                    def _():
                        fetch_wait(0, 2 * i + 2)
                        gath(buf_a, sga, 0, 0)
                else:
                    gath(buf_a, sga, par2, k2)

                gath_wait(buf_b, sgb, par, kb)
                scat(buf_b, ssb, par, kb)

                if m == 4:
                    @pl.when(2 * i + 2 < n_win)
                    def _():
                        fetch_win(0, 2 * i + 2)
            return carry

        lax.fori_loop(0, n_win // 2, body, 0)
        scat_wait(buf_b, ssb, 1, 7)
        plsc.subcore_barrier()
        pltpu.sync_copy(acc_s.at[pl.ds(row0, STRIPE)],
                        p_hbm.at[c].at[pl.ds(row0, STRIPE)])

    return agg(h, src3, dst3, z128)


BLK = 400


def _tc_layer(p, d, h, Wl, bl, Wr, relu):
    nb = N // BLK

    def body(p_ref, d_ref, h_ref, wl_ref, bl_ref, wr_ref, o_ref):
        deg = d_ref[0, :, 0:1] + d_ref[1, :, 0:1]
        mean = (p_ref[0] + p_ref[1]) / jnp.maximum(deg, 1.0)
        out = (lax.dot_general(mean, wl_ref[...], (((1,), (1,)), ((), ())),
                               preferred_element_type=jnp.float32)
               + lax.dot_general(h_ref[...], wr_ref[...],
                                 (((1,), (1,)), ((), ())),
                                 preferred_element_type=jnp.float32)
               + bl_ref[...])
        if relu:
            out = jnp.maximum(out, 0.0)
        o_ref[...] = out

    return pl.pallas_call(
        body,
        grid=(nb,),
        in_specs=[
            pl.BlockSpec((NC, BLK, D), lambda i: (0, i, 0)),
            pl.BlockSpec((NC, BLK, D), lambda i: (0, i, 0)),
            pl.BlockSpec((BLK, D), lambda i: (i, 0)),
            pl.BlockSpec((D, D), lambda i: (0, 0)),
            pl.BlockSpec((1, D), lambda i: (0, 0)),
            pl.BlockSpec((D, D), lambda i: (0, 0)),
        ],
        out_specs=pl.BlockSpec((BLK, D), lambda i: (i, 0)),
        out_shape=jax.ShapeDtypeStruct((N, D), jnp.float32),
    )(p, d, h, Wl, bl, Wr)


def kernel(x, edge_index, Wl1, bl1, Wr1, Wl2, bl2, Wr2, Wl3, bl3, Wr3):
    src = edge_index[0]
    dst = edge_index[1]
    e = src.shape[0]
    n_chunks = -(-e // (NW * CHUNK))
    n_chunks = -(-n_chunks // 16) * 16
    e_pad = NW * CHUNK * n_chunks
    pad = e_pad - e
    src_p = jnp.concatenate(
        [src, jnp.zeros((pad,), jnp.int32)]).reshape(NW, n_chunks, CHUNK)
    dst_p = jnp.concatenate(
        [dst, jnp.full((pad,), N, jnp.int32)]).reshape(NW, n_chunks, CHUNK)
    z128 = jnp.zeros((STRIPE, D), jnp.float32)
    ones128 = jnp.ones((CHUNK, D), jnp.float32)

    dg = _sc_degree(dst_p, z128, ones128, n_chunks)[:, :N]

    def layer(h, Wl, bl, Wr, relu):
        p = _sc_aggregate(h, src_p, dst_p, z128, n_chunks)
        return _tc_layer(p[:, :N], dg, h, Wl, bl.reshape(1, D), Wr, relu)

    h = layer(x, Wl1, bl1, Wr1, True)
    h = layer(h, Wl2, bl2, Wr2, True)
    return layer(h, Wl3, bl3, Wr3, False)
